# baseline (speedup 1.0000x reference)
import functools

import jax
import jax.numpy as jnp
from jax.experimental import pallas as pl
from jax.experimental.pallas import tpu as pltpu
from jax.experimental.pallas import tpu_sc as plsc

V = 100000
EMB = 400
H = 1050
NL = 4
B = 32
T = 32
NTOK = B * T

_GATHER_W = 128


def _sc_embed_gather(E, idx_flat):
    mesh = plsc.VectorSubcoreMesh(core_axis_name="c", subcore_axis_name="s")
    NW = 32
    bpw = NTOK // NW

    @functools.partial(
        pl.kernel,
        out_type=jax.ShapeDtypeStruct((NTOK, EMB), jnp.float32),
        mesh=mesh,
        scratch_types=[
            pltpu.VMEM((bpw,), jnp.int32),
            pltpu.VMEM((bpw, EMB), jnp.float32),
            pltpu.SemaphoreType.DMA,
        ],
        compiler_params=pltpu.CompilerParams(use_tc_tiling_on_sc=False),
    )
    def gather_kernel(e_hbm, i_hbm, o_hbm, idx_v, rows_v, sem):
        wid = jax.lax.axis_index("s") * 2 + jax.lax.axis_index("c")
        base = wid * bpw
        pltpu.sync_copy(i_hbm.at[pl.ds(base, bpw)], idx_v)
        pltpu.async_copy(e_hbm.at[idx_v], rows_v, sem).wait()
        pltpu.sync_copy(rows_v, o_hbm.at[pl.ds(base, bpw)])

    return gather_kernel(E, idx_flat)


def _gx_kernel(x_ref, w_ref, b_ref, gx_ref):
    gx_ref[0] = (
        jnp.dot(x_ref[...], w_ref[0], preferred_element_type=jnp.float32)
        + b_ref[0]
    )


def _input_proj(xs, WihT, bias):
    din = xs.shape[1]
    return pl.pallas_call(
        _gx_kernel,
        grid=(4,),
        in_specs=[
            pl.BlockSpec((NTOK, din), lambda g: (0, 0)),
            pl.BlockSpec((1, din, H), lambda g: (g, 0, 0)),
            pl.BlockSpec((1, 1, H), lambda g: (g, 0, 0)),
        ],
        out_specs=pl.BlockSpec((1, NTOK, H), lambda g: (g, 0, 0)),
        out_shape=jax.ShapeDtypeStruct((4, NTOK, H), jnp.float32),
    )(xs, WihT, bias)


def _rec_kernel(gx_ref, w_ref, h0_ref, c0_ref, ys_ref, hT_ref, cT_ref, h_scr, c_scr):
    t = pl.program_id(0)

    @pl.when(t == 0)
    def _():
        h_scr[...] = h0_ref[...]
        c_scr[...] = c0_ref[...]

    h = h_scr[...]
    gi = jax.nn.sigmoid(
        gx_ref[0] + jnp.dot(h, w_ref[0], preferred_element_type=jnp.float32)
    )
    gf = jax.nn.sigmoid(
        gx_ref[1] + jnp.dot(h, w_ref[1], preferred_element_type=jnp.float32)
    )
    gg = jnp.tanh(
        gx_ref[2] + jnp.dot(h, w_ref[2], preferred_element_type=jnp.float32)
    )
    go = jax.nn.sigmoid(
        gx_ref[3] + jnp.dot(h, w_ref[3], preferred_element_type=jnp.float32)
    )
    c = gf * c_scr[...] + gi * gg
    h = go * jnp.tanh(c)
    h_scr[...] = h
    c_scr[...] = c
    ys_ref[...] = h

    @pl.when(t == T - 1)
    def _():
        hT_ref[...] = h
        cT_ref[...] = c


def _recurrent(gx, WhhT, h0, c0):
    return pl.pallas_call(
        _rec_kernel,
        grid=(T,),
        in_specs=[
            pl.BlockSpec((4, B, H), lambda t: (0, t, 0)),
            pl.BlockSpec((4, H, H), lambda t: (0, 0, 0)),
            pl.BlockSpec((B, H), lambda t: (0, 0)),
            pl.BlockSpec((B, H), lambda t: (0, 0)),
        ],
        out_specs=[
            pl.BlockSpec((B, H), lambda t: (t, 0)),
            pl.BlockSpec((B, H), lambda t: (0, 0)),
            pl.BlockSpec((B, H), lambda t: (0, 0)),
        ],
        out_shape=[
            jax.ShapeDtypeStruct((NTOK, H), jnp.float32),
            jax.ShapeDtypeStruct((B, H), jnp.float32),
            jax.ShapeDtypeStruct((B, H), jnp.float32),
        ],
        scratch_shapes=[
            pltpu.VMEM((B, H), jnp.float32),
            pltpu.VMEM((B, H), jnp.float32),
        ],
        compiler_params=pltpu.CompilerParams(
            dimension_semantics=("arbitrary",),
        ),
    )(gx, WhhT, h0, c0)


_BV = 2048
_NV = (V + _BV - 1) // _BV


def _proj_kernel(x_ref, w_ref, b_ref, o_ref):
    o_ref[...] = (
        jax.lax.dot_general(
            x_ref[...],
            w_ref[...],
            (((1,), (1,)), ((), ())),
            preferred_element_type=jnp.float32,
        )
        + b_ref[...]
    )


def _vocab_proj(out_bt, W_out, b_out):
    b2 = b_out.reshape(1, V)
    return pl.pallas_call(
        _proj_kernel,
        grid=(_NV,),
        in_specs=[
            pl.BlockSpec((NTOK, H), lambda v: (0, 0)),
            pl.BlockSpec((_BV, H), lambda v: (v, 0)),
            pl.BlockSpec((1, _BV), lambda v: (0, v)),
        ],
        out_specs=pl.BlockSpec((NTOK, _BV), lambda v: (0, v)),
        out_shape=jax.ShapeDtypeStruct((NTOK, V), jnp.float32),
    )(out_bt, W_out, b2)


def _lstm_stack(xs, h0, c0, layer_params):
    hs = []
    cs = []
    for l, (WihT, WhhT, bias) in enumerate(layer_params):
        gx = _input_proj(xs, WihT, bias)
        xs, hT, cT = _recurrent(gx, WhhT, h0[l], c0[l])
        hs.append(hT)
        cs.append(cT)
    return xs, jnp.stack(hs), jnp.stack(cs)


def kernel(x, h0, c0, E, W_ih0, W_hh0, b_ih0, b_hh0, W_ih1, W_hh1, b_ih1, b_hh1, W_ih2, W_hh2, b_ih2, b_hh2, W_ih3, W_hh3, b_ih3, b_hh3, W_out, b_out):
    idx = x.astype(jnp.int32).T.reshape(NTOK)
    xs = _sc_embed_gather(E, idx)

    layer_params = []
    for W_ih, W_hh, b_ih, b_hh in (
        (W_ih0, W_hh0, b_ih0, b_hh0),
        (W_ih1, W_hh1, b_ih1, b_hh1),
        (W_ih2, W_hh2, b_ih2, b_hh2),
        (W_ih3, W_hh3, b_ih3, b_hh3),
    ):
        din = W_ih.shape[1]
        WihT = W_ih.reshape(4, H, din).transpose(0, 2, 1)
        WhhT = W_hh.reshape(4, H, H).transpose(0, 2, 1)
        bias = (b_ih + b_hh).reshape(4, 1, H)
        layer_params.append((WihT, WhhT, bias))

    ys, hs, cs = _lstm_stack(xs, h0, c0, layer_params)

    out_bt = ys.reshape(T, B, H).transpose(1, 0, 2).reshape(NTOK, H)
    logits = _vocab_proj(out_bt, W_out, b_out)
    return logits, (hs, cs)


# speedup vs baseline: 1.3883x; 1.3883x over previous
import functools

import jax
import jax.numpy as jnp
from jax import lax
from jax.experimental import pallas as pl
from jax.experimental.pallas import tpu as pltpu
from jax.experimental.pallas import tpu_sc as plsc

V = 100000
EMB = 400
H = 1050
NL = 4
B = 32
T = 32
NTOK = B * T


def _sc_embed_gather(E, idx_flat):
    mesh = plsc.ScalarSubcoreMesh(axis_name="core", num_cores=2)
    npc = NTOK // 2
    pieces = ((0, 128), (128, 128), (256, 128), (384, 16))

    @functools.partial(
        pl.kernel,
        out_type=jax.ShapeDtypeStruct((NTOK, EMB), jnp.float32),
        mesh=mesh,
        scratch_types=[
            pltpu.SMEM((npc,), jnp.int32),
            pltpu.SemaphoreType.DMA,
        ],
    )
    def gather_kernel(e_hbm, i_hbm, o_hbm, idx_s, sem):
        cid = lax.axis_index("core")
        base = cid * npc
        pltpu.async_copy(i_hbm.at[pl.ds(base, npc)], idx_s, sem).wait()

        @pl.loop(0, npc)
        def _(j):
            tok = idx_s[j]
            row = base + j
            for off, w in pieces:
                pltpu.async_copy(
                    e_hbm.at[tok, pl.ds(off, w)],
                    o_hbm.at[row, pl.ds(off, w)],
                    sem,
                )

        @pl.loop(0, npc)
        def _(j):
            row = base + j
            for off, w in pieces:
                pltpu.make_async_copy(
                    e_hbm.at[0, pl.ds(off, w)],
                    o_hbm.at[row, pl.ds(off, w)],
                    sem,
                ).wait()

    return gather_kernel(E, idx_flat)


def _gx_kernel(x_ref, w_ref, b_ref, gx_ref):
    gx_ref[0] = (
        lax.dot_general(
            x_ref[...],
            w_ref[0],
            (((1,), (1,)), ((), ())),
            preferred_element_type=jnp.float32,
        )
        + b_ref[0]
    )


def _input_proj(xs, Wih, bias):
    din = xs.shape[1]
    return pl.pallas_call(
        _gx_kernel,
        grid=(4,),
        in_specs=[
            pl.BlockSpec((NTOK, din), lambda g: (0, 0)),
            pl.BlockSpec((1, H, din), lambda g: (g, 0, 0)),
            pl.BlockSpec((1, 1, H), lambda g: (g, 0, 0)),
        ],
        out_specs=pl.BlockSpec((1, NTOK, H), lambda g: (g, 0, 0)),
        out_shape=jax.ShapeDtypeStruct((4, NTOK, H), jnp.float32),
    )(xs, Wih, bias)


def _rec_kernel(gx_ref, w_ref, h0_ref, c0_ref, ys_ref, hT_ref, cT_ref,
                wt_scr, h_scr, c_scr):
    t = pl.program_id(0)

    @pl.when(t == 0)
    def _():
        for g in range(4):
            wt_scr[g] = w_ref[g].T
        h_scr[...] = h0_ref[...]
        c_scr[...] = c0_ref[...]

    h = h_scr[...]
    gi = jax.nn.sigmoid(
        gx_ref[0] + jnp.dot(h, wt_scr[0], preferred_element_type=jnp.float32)
    )
    gf = jax.nn.sigmoid(
        gx_ref[1] + jnp.dot(h, wt_scr[1], preferred_element_type=jnp.float32)
    )
    gg = jnp.tanh(
        gx_ref[2] + jnp.dot(h, wt_scr[2], preferred_element_type=jnp.float32)
    )
    go = jax.nn.sigmoid(
        gx_ref[3] + jnp.dot(h, wt_scr[3], preferred_element_type=jnp.float32)
    )
    c = gf * c_scr[...] + gi * gg
    h = go * jnp.tanh(c)
    h_scr[...] = h
    c_scr[...] = c
    ys_ref[...] = h

    @pl.when(t == T - 1)
    def _():
        hT_ref[...] = h
        cT_ref[...] = c


def _recurrent(gx, Whh, h0, c0):
    ys_shape = jax.ShapeDtypeStruct((NTOK, H), jnp.float32)
    ys_spec = pl.BlockSpec((B, H), lambda t: (t, 0))
    return pl.pallas_call(
        _rec_kernel,
        grid=(T,),
        in_specs=[
            pl.BlockSpec((4, B, H), lambda t: (0, t, 0)),
            pl.BlockSpec((4, H, H), lambda t: (0, 0, 0)),
            pl.BlockSpec((B, H), lambda t: (0, 0)),
            pl.BlockSpec((B, H), lambda t: (0, 0)),
        ],
        out_specs=[
            ys_spec,
            pl.BlockSpec((B, H), lambda t: (0, 0)),
            pl.BlockSpec((B, H), lambda t: (0, 0)),
        ],
        out_shape=[
            ys_shape,
            jax.ShapeDtypeStruct((B, H), jnp.float32),
            jax.ShapeDtypeStruct((B, H), jnp.float32),
        ],
        scratch_shapes=[
            pltpu.VMEM((4, H, H), jnp.float32),
            pltpu.VMEM((B, H), jnp.float32),
            pltpu.VMEM((B, H), jnp.float32),
        ],
        compiler_params=pltpu.CompilerParams(
            dimension_semantics=("arbitrary",),
        ),
    )(gx, Whh, h0, c0)


_BV = 2048
_NV = (V + _BV - 1) // _BV


def _proj_kernel(ys_ref, w_ref, b_ref, o_ref, x_scr):
    v = pl.program_id(0)

    @pl.when(v == 0)
    def _():
        ys = ys_ref[...].reshape(T, B, H)
        x_scr[...] = jnp.swapaxes(ys, 0, 1).reshape(NTOK, H)

    o_ref[...] = (
        lax.dot_general(
            x_scr[...],
            w_ref[...],
            (((1,), (1,)), ((), ())),
            preferred_element_type=jnp.float32,
        )
        + b_ref[...]
    )


def _vocab_proj(ys, W_out, b_out):
    b2 = b_out.reshape(1, V)
    return pl.pallas_call(
        _proj_kernel,
        grid=(_NV,),
        in_specs=[
            pl.BlockSpec((NTOK, H), lambda v: (0, 0)),
            pl.BlockSpec((_BV, H), lambda v: (v, 0)),
            pl.BlockSpec((1, _BV), lambda v: (0, v)),
        ],
        out_specs=pl.BlockSpec((NTOK, _BV), lambda v: (0, v)),
        out_shape=jax.ShapeDtypeStruct((NTOK, V), jnp.float32),
        scratch_shapes=[pltpu.VMEM((NTOK, H), jnp.float32)],
        compiler_params=pltpu.CompilerParams(
            dimension_semantics=("arbitrary",),
        ),
    )(ys, W_out, b2)


def kernel(x, h0, c0, E, W_ih0, W_hh0, b_ih0, b_hh0, W_ih1, W_hh1, b_ih1, b_hh1, W_ih2, W_hh2, b_ih2, b_hh2, W_ih3, W_hh3, b_ih3, b_hh3, W_out, b_out):
    idx = x.astype(jnp.int32).T.reshape(NTOK)
    xs = _sc_embed_gather(E, idx)

    layer_params = []
    for W_ih, W_hh, b_ih, b_hh in (
        (W_ih0, W_hh0, b_ih0, b_hh0),
        (W_ih1, W_hh1, b_ih1, b_hh1),
        (W_ih2, W_hh2, b_ih2, b_hh2),
        (W_ih3, W_hh3, b_ih3, b_hh3),
    ):
        din = W_ih.shape[1]
        layer_params.append((
            W_ih.reshape(4, H, din),
            W_hh.reshape(4, H, H),
            (b_ih + b_hh).reshape(4, 1, H),
        ))

    hs = []
    cs = []
    for l, (Wih, Whh, bias) in enumerate(layer_params):
        gx = _input_proj(xs, Wih, bias)
        xs, hT, cT = _recurrent(gx, Whh, h0[l], c0[l])
        hs.append(hT)
        cs.append(cT)

    logits = _vocab_proj(xs, W_out, b_out)
    return logits, (jnp.stack(hs), jnp.stack(cs))


# speedup vs baseline: 2.9912x; 2.1547x over previous
import functools

import jax
import jax.numpy as jnp
from jax import lax
from jax.experimental import pallas as pl
from jax.experimental.pallas import tpu as pltpu
from jax.experimental.pallas import tpu_sc as plsc

V = 100000
EMB = 400
H = 1050
NL = 4
B = 32
T = 32
NTOK = B * T
G4 = 4 * H


def _sc_embed_gather(E_rowmajor, idx_flat):
    mesh = plsc.ScalarSubcoreMesh(axis_name="core", num_cores=2)
    npc = NTOK // 2
    pieces = ((0, 128), (128, 128), (256, 128), (384, 16))

    @functools.partial(
        pl.kernel,
        out_type=jax.ShapeDtypeStruct((NTOK, EMB), jnp.float32),
        mesh=mesh,
        scratch_types=[
            pltpu.SMEM((npc,), jnp.int32),
            pltpu.SemaphoreType.DMA,
        ],
    )
    def gather_kernel(e_hbm, i_hbm, o_hbm, idx_s, sem):
        cid = lax.axis_index("core")
        base = cid * npc
        pltpu.async_copy(i_hbm.at[pl.ds(base, npc)], idx_s, sem).wait()

        @pl.loop(0, npc)
        def _(j):
            tok = idx_s[j]
            row = base + j
            for off, w in pieces:
                pltpu.async_copy(
                    e_hbm.at[tok, pl.ds(off, w)],
                    o_hbm.at[row, pl.ds(off, w)],
                    sem,
                )

        @pl.loop(0, npc)
        def _(j):
            row = base + j
            for off, w in pieces:
                pltpu.make_async_copy(
                    e_hbm.at[0, pl.ds(off, w)],
                    o_hbm.at[row, pl.ds(off, w)],
                    sem,
                ).wait()

    return gather_kernel(E_rowmajor, idx_flat)


def _gx_kernel(x_ref, w_ref, b_ref, gx_ref):
    gx_ref[...] = (
        jnp.dot(x_ref[...], w_ref[...], preferred_element_type=jnp.float32)
        + b_ref[...]
    )


_BG = 1024
_NG = (G4 + _BG - 1) // _BG


def _input_proj(xs, WihT, bias2d):
    din = xs.shape[1]
    return pl.pallas_call(
        _gx_kernel,
        grid=(_NG,),
        in_specs=[
            pl.BlockSpec((NTOK, din), lambda g: (0, 0)),
            pl.BlockSpec((din, _BG), lambda g: (0, g)),
            pl.BlockSpec((1, _BG), lambda g: (0, g)),
        ],
        out_specs=pl.BlockSpec((NTOK, _BG), lambda g: (0, g)),
        out_shape=jax.ShapeDtypeStruct((NTOK, G4), jnp.float32),
    )(xs, WihT, bias2d)


def _rec_kernel(gx_ref, w_ref, h0_ref, c0_ref, ys_ref, hT_ref, cT_ref,
                h_scr, c_scr):
    t = pl.program_id(0)

    @pl.when(t == 0)
    def _():
        h_scr[...] = h0_ref[...]
        c_scr[...] = c0_ref[...]

    h = h_scr[...]
    g = gx_ref[...] + jnp.dot(h, w_ref[...], preferred_element_type=jnp.float32)
    gi = jax.nn.sigmoid(g[:, 0:H])
    gf = jax.nn.sigmoid(g[:, H:2 * H])
    gg = jnp.tanh(g[:, 2 * H:3 * H])
    go = jax.nn.sigmoid(g[:, 3 * H:4 * H])
    c = gf * c_scr[...] + gi * gg
    h = go * jnp.tanh(c)
    h_scr[...] = h
    c_scr[...] = c
    ys_ref[...] = h

    @pl.when(t == T - 1)
    def _():
        hT_ref[...] = h
        cT_ref[...] = c


def _recurrent(gx, WhhT, h0, c0):
    return pl.pallas_call(
        _rec_kernel,
        grid=(T,),
        in_specs=[
            pl.BlockSpec((B, G4), lambda t: (t, 0)),
            pl.BlockSpec((H, G4), lambda t: (0, 0)),
            pl.BlockSpec((B, H), lambda t: (0, 0)),
            pl.BlockSpec((B, H), lambda t: (0, 0)),
        ],
        out_specs=[
            pl.BlockSpec((B, H), lambda t: (t, 0)),
            pl.BlockSpec((B, H), lambda t: (0, 0)),
            pl.BlockSpec((B, H), lambda t: (0, 0)),
        ],
        out_shape=[
            jax.ShapeDtypeStruct((NTOK, H), jnp.float32),
            jax.ShapeDtypeStruct((B, H), jnp.float32),
            jax.ShapeDtypeStruct((B, H), jnp.float32),
        ],
        scratch_shapes=[
            pltpu.VMEM((B, H), jnp.float32),
            pltpu.VMEM((B, H), jnp.float32),
        ],
        compiler_params=pltpu.CompilerParams(
            dimension_semantics=("arbitrary",),
        ),
    )(gx, WhhT, h0, c0)


_BV = 2048
_NV = (V + _BV - 1) // _BV


def _proj_kernel(ys_ref, w_ref, b_ref, o_ref, x_scr):
    v = pl.program_id(0)

    @pl.when(v == 0)
    def _():
        ys = ys_ref[...].reshape(T, B, H)
        x_scr[...] = jnp.swapaxes(ys, 0, 1).reshape(NTOK, H)

    p = jnp.dot(x_scr[...], w_ref[...], preferred_element_type=jnp.float32)
    o_ref[...] = p.T + b_ref[...]


def _vocab_proj(ys, WoutT, b_col):
    return pl.pallas_call(
        _proj_kernel,
        grid=(_NV,),
        in_specs=[
            pl.BlockSpec((NTOK, H), lambda v: (0, 0)),
            pl.BlockSpec((H, _BV), lambda v: (0, v)),
            pl.BlockSpec((_BV, 1), lambda v: (v, 0)),
        ],
        out_specs=pl.BlockSpec((_BV, NTOK), lambda v: (v, 0)),
        out_shape=jax.ShapeDtypeStruct((V, NTOK), jnp.float32),
        scratch_shapes=[pltpu.VMEM((NTOK, H), jnp.float32)],
        compiler_params=pltpu.CompilerParams(
            dimension_semantics=("arbitrary",),
        ),
    )(ys, WoutT, b_col)


def kernel(x, h0, c0, E, W_ih0, W_hh0, b_ih0, b_hh0, W_ih1, W_hh1, b_ih1, b_hh1, W_ih2, W_hh2, b_ih2, b_hh2, W_ih3, W_hh3, b_ih3, b_hh3, W_out, b_out):
    idx = x.astype(jnp.int32).T.reshape(NTOK)
    xs = _sc_embed_gather(E, idx)

    hs = []
    cs = []
    for l, (W_ih, W_hh, b_ih, b_hh) in enumerate((
        (W_ih0, W_hh0, b_ih0, b_hh0),
        (W_ih1, W_hh1, b_ih1, b_hh1),
        (W_ih2, W_hh2, b_ih2, b_hh2),
        (W_ih3, W_hh3, b_ih3, b_hh3),
    )):
        gx = _input_proj(xs, W_ih.T, (b_ih + b_hh).reshape(1, G4))
        xs, hT, cT = _recurrent(gx, W_hh.T, h0[l], c0[l])
        hs.append(hT)
        cs.append(cT)

    logits_t = _vocab_proj(xs, W_out.T, b_out.reshape(V, 1))
    return logits_t.T, (jnp.stack(hs), jnp.stack(cs))


# speedup vs baseline: 3.0028x; 1.0039x over previous
import functools

import jax
import jax.numpy as jnp
from jax import lax
from jax.experimental import pallas as pl
from jax.experimental.pallas import tpu as pltpu
from jax.experimental.pallas import tpu_sc as plsc

V = 100000
EMB = 400
H = 1050
NL = 4
B = 32
T = 32
NTOK = B * T
G4 = 4 * H


def _sc_embed_gather(E_rowmajor, idx_flat):
    mesh = plsc.ScalarSubcoreMesh(axis_name="core", num_cores=2)
    npc = NTOK // 2
    pieces = ((0, 128), (128, 128), (256, 128), (384, 16))

    @functools.partial(
        pl.kernel,
        out_type=jax.ShapeDtypeStruct((NTOK, EMB), jnp.float32),
        mesh=mesh,
        scratch_types=[
            pltpu.SMEM((npc,), jnp.int32),
            pltpu.SemaphoreType.DMA,
        ],
    )
    def gather_kernel(e_hbm, i_hbm, o_hbm, idx_s, sem):
        cid = lax.axis_index("core")
        base = cid * npc
        pltpu.async_copy(i_hbm.at[pl.ds(base, npc)], idx_s, sem).wait()

        @pl.loop(0, npc)
        def _(j):
            tok = idx_s[j]
            row = base + j
            for off, w in pieces:
                pltpu.async_copy(
                    e_hbm.at[tok, pl.ds(off, w)],
                    o_hbm.at[row, pl.ds(off, w)],
                    sem,
                )

        @pl.loop(0, npc)
        def _(j):
            row = base + j
            for off, w in pieces:
                pltpu.make_async_copy(
                    e_hbm.at[0, pl.ds(off, w)],
                    o_hbm.at[row, pl.ds(off, w)],
                    sem,
                ).wait()

    return gather_kernel(E_rowmajor, idx_flat)


_BG = 1024
_NG = (G4 + _BG - 1) // _BG
_G4P = _NG * _BG


def _layer_kernel(x_ref, wih_ref, whh_ref, b_ref, h0_ref, c0_ref,
                  ys_ref, hT_ref, cT_ref, gx_scr, h_scr, c_scr):
    j = pl.program_id(0)

    @pl.when(j < _NG)
    def _():
        gx_scr[:, pl.ds(j * _BG, _BG)] = (
            jnp.dot(x_ref[...], wih_ref[...], preferred_element_type=jnp.float32)
            + b_ref[...]
        )

    @pl.when(j == _NG)
    def _():
        h_scr[...] = h0_ref[...]
        c_scr[...] = c0_ref[...]

    @pl.when(j >= _NG)
    def _():
        t = j - _NG
        h = h_scr[...]
        g = gx_scr[pl.ds(t * B, B), 0:G4] + jnp.dot(
            h, whh_ref[...], preferred_element_type=jnp.float32
        )
        gi = jax.nn.sigmoid(g[:, 0:H])
        gf = jax.nn.sigmoid(g[:, H:2 * H])
        gg = jnp.tanh(g[:, 2 * H:3 * H])
        go = jax.nn.sigmoid(g[:, 3 * H:4 * H])
        c = gf * c_scr[...] + gi * gg
        h = go * jnp.tanh(c)
        h_scr[...] = h
        c_scr[...] = c
        ys_ref[...] = h

        @pl.when(t == T - 1)
        def _():
            hT_ref[...] = h
            cT_ref[...] = c


def _lstm_layer(xs, WihT, WhhT, bias2d, h0, c0):
    din = xs.shape[1]
    return pl.pallas_call(
        _layer_kernel,
        grid=(_NG + T,),
        in_specs=[
            pl.BlockSpec((NTOK, din), lambda j: (0, 0)),
            pl.BlockSpec((din, _BG), lambda j: (0, jnp.minimum(j, _NG - 1))),
            pl.BlockSpec((H, G4), lambda j: (0, 0)),
            pl.BlockSpec((1, _BG), lambda j: (0, jnp.minimum(j, _NG - 1))),
            pl.BlockSpec((B, H), lambda j: (0, 0)),
            pl.BlockSpec((B, H), lambda j: (0, 0)),
        ],
        out_specs=[
            pl.BlockSpec((B, H), lambda j: (jnp.maximum(j - _NG, 0), 0)),
            pl.BlockSpec((B, H), lambda j: (0, 0)),
            pl.BlockSpec((B, H), lambda j: (0, 0)),
        ],
        out_shape=[
            jax.ShapeDtypeStruct((NTOK, H), jnp.float32),
            jax.ShapeDtypeStruct((B, H), jnp.float32),
            jax.ShapeDtypeStruct((B, H), jnp.float32),
        ],
        scratch_shapes=[
            pltpu.VMEM((NTOK, _G4P), jnp.float32),
            pltpu.VMEM((B, H), jnp.float32),
            pltpu.VMEM((B, H), jnp.float32),
        ],
        compiler_params=pltpu.CompilerParams(
            dimension_semantics=("arbitrary",),
        ),
    )(xs, WihT, WhhT, bias2d, h0, c0)


_BV = 2048
_NV = (V + _BV - 1) // _BV


def _proj_kernel(ys_ref, w_ref, b_ref, o_ref, x_scr):
    v = pl.program_id(0)

    @pl.when(v == 0)
    def _():
        ys = ys_ref[...].reshape(T, B, H)
        x_scr[...] = jnp.swapaxes(ys, 0, 1).reshape(NTOK, H)

    p = jnp.dot(x_scr[...], w_ref[...], preferred_element_type=jnp.float32)
    o_ref[...] = p.T + b_ref[...]


def _vocab_proj(ys, WoutT, b_col):
    return pl.pallas_call(
        _proj_kernel,
        grid=(_NV,),
        in_specs=[
            pl.BlockSpec((NTOK, H), lambda v: (0, 0)),
            pl.BlockSpec((H, _BV), lambda v: (0, v)),
            pl.BlockSpec((_BV, 1), lambda v: (v, 0)),
        ],
        out_specs=pl.BlockSpec((_BV, NTOK), lambda v: (v, 0)),
        out_shape=jax.ShapeDtypeStruct((V, NTOK), jnp.float32),
        scratch_shapes=[pltpu.VMEM((NTOK, H), jnp.float32)],
        compiler_params=pltpu.CompilerParams(
            dimension_semantics=("arbitrary",),
        ),
    )(ys, WoutT, b_col)


def kernel(x, h0, c0, E, W_ih0, W_hh0, b_ih0, b_hh0, W_ih1, W_hh1, b_ih1, b_hh1, W_ih2, W_hh2, b_ih2, b_hh2, W_ih3, W_hh3, b_ih3, b_hh3, W_out, b_out):
    idx = x.astype(jnp.int32).T.reshape(NTOK)
    xs = _sc_embed_gather(E, idx)

    hs = []
    cs = []
    for l, (W_ih, W_hh, b_ih, b_hh) in enumerate((
        (W_ih0, W_hh0, b_ih0, b_hh0),
        (W_ih1, W_hh1, b_ih1, b_hh1),
        (W_ih2, W_hh2, b_ih2, b_hh2),
        (W_ih3, W_hh3, b_ih3, b_hh3),
    )):
        xs, hT, cT = _lstm_layer(
            xs, W_ih.T, W_hh.T, (b_ih + b_hh).reshape(1, G4), h0[l], c0[l]
        )
        hs.append(hT)
        cs.append(cT)

    logits_t = _vocab_proj(xs, W_out.T, b_out.reshape(V, 1))
    return logits_t.T, (jnp.stack(hs), jnp.stack(cs))
